# baseline (speedup 1.0000x reference)
import functools

import jax
import jax.numpy as jnp
from jax import lax
from jax.experimental import pallas as pl
from jax.experimental.pallas import tpu as pltpu
from jax.experimental.pallas import tpu_sc as plsc

_VOCAB = 1000000
_DIM = 32
_B = 4096
_C = 4096


@functools.lru_cache(maxsize=None)
def _make_sc_gather(V, D, B, C):
    NC, NS = 2, 16
    NW = NC * NS
    b_per_w = B // NW
    c_per_w = C // NW
    mesh = plsc.VectorSubcoreMesh(core_axis_name="c", subcore_axis_name="s")

    @functools.partial(
        pl.kernel,
        mesh=mesh,
        compiler_params=pltpu.CompilerParams(use_tc_tiling_on_sc=False),
        out_type=[
            jax.ShapeDtypeStruct((B, D), jnp.float32),
            jax.ShapeDtypeStruct((C, D), jnp.float32),
        ],
        scratch_types=[
            pltpu.VMEM((b_per_w,), jnp.int32),
            pltpu.VMEM((b_per_w, D), jnp.float32),
            pltpu.VMEM((c_per_w,), jnp.int32),
            pltpu.VMEM((c_per_w, D), jnp.float32),
            pltpu.SemaphoreType.DMA,
        ],
    )
    def gather_k(win_hbm, cidx_hbm, wout_hbm, xidx_hbm, outc_hbm, outx_hbm,
                 cidx_v, crows_v, xidx_v, xrows_v, sem):
        wid = lax.axis_index("s") * NC + lax.axis_index("c")
        cbase = wid * b_per_w
        xbase = wid * c_per_w
        pltpu.sync_copy(cidx_hbm.at[pl.ds(cbase, b_per_w)], cidx_v)
        pltpu.sync_copy(xidx_hbm.at[pl.ds(xbase, c_per_w)], xidx_v)
        cp1 = pltpu.async_copy(win_hbm.at[cidx_v], crows_v, sem)
        cp2 = pltpu.async_copy(wout_hbm.at[xidx_v], xrows_v, sem)
        cp1.wait()
        cp2.wait()
        pltpu.sync_copy(crows_v, outc_hbm.at[pl.ds(cbase, b_per_w)])
        pltpu.sync_copy(xrows_v, outx_hbm.at[pl.ds(xbase, c_per_w)])

    return gather_k


def _mm_body(cv_ref, xv_ref, out_ref):
    out_ref[...] = lax.dot_general(
        cv_ref[...], xv_ref[...],
        (((1,), (1,)), ((), ())),
        preferred_element_type=jnp.float32,
    )


def _matmul(cv, xv):
    BM = 512
    BN = 4096
    grid = (_B // BM, _C // BN)
    return pl.pallas_call(
        _mm_body,
        grid=grid,
        in_specs=[
            pl.BlockSpec((BM, _DIM), lambda i, j: (i, 0)),
            pl.BlockSpec((BN, _DIM), lambda i, j: (j, 0)),
        ],
        out_specs=pl.BlockSpec((BM, BN), lambda i, j: (i, j)),
        out_shape=jax.ShapeDtypeStruct((_B, _C), jnp.float32),
    )(cv, xv)


def kernel(center_words, all_context_words, W_in, W_out):
    cidx = center_words.astype(jnp.int32)
    xidx = all_context_words.astype(jnp.int32)
    cv, xv = _make_sc_gather(_VOCAB, _DIM, _B, _C)(W_in, cidx, W_out, xidx)
    return _matmul(cv, xv)


# speedup vs baseline: 3.0137x; 3.0137x over previous
import functools

import jax
import jax.numpy as jnp
from jax import lax
from jax.experimental import pallas as pl
from jax.experimental.pallas import tpu as pltpu
from jax.experimental.pallas import tpu_sc as plsc

_VOCAB = 1000000
_DIM = 32
_B = 4096
_C = 4096


@functools.lru_cache(maxsize=None)
def _make_sc_gather(V, D, B, C):
    NC, NS = 2, 16
    NW = NC * NS
    b_per_w = B // NW
    c_per_w = C // NW
    DH = D // 8
    mesh = plsc.VectorSubcoreMesh(core_axis_name="c", subcore_axis_name="s")

    @functools.partial(
        pl.kernel,
        mesh=mesh,
        out_type=[
            jax.ShapeDtypeStruct((DH, 8, B), jnp.float32),
            jax.ShapeDtypeStruct((DH, 8, C), jnp.float32),
        ],
        scratch_types=[
            pltpu.VMEM((b_per_w,), jnp.int32),
            pltpu.VMEM((b_per_w,), jnp.int32),
            pltpu.VMEM((c_per_w,), jnp.int32),
            pltpu.VMEM((c_per_w,), jnp.int32),
            pltpu.VMEM((DH, 8, 128), jnp.float32),
            pltpu.VMEM((DH, 8, 128), jnp.float32),
            pltpu.VMEM((DH, 8, b_per_w), jnp.float32),
            pltpu.VMEM((DH, 8, c_per_w), jnp.float32),
            pltpu.SemaphoreType.DMA,
        ],
    )
    def gather_k(winT_hbm, ctcol_hbm, clane_hbm, woutT_hbm, xtcol_hbm,
                 xlane_hbm, outcT_hbm, outxT_hbm,
                 ctcol_v, clane_v, xtcol_v, xlane_v,
                 ctile_v, xtile_v, cacc_v, xacc_v, sem):
        wid = lax.axis_index("s") * NC + lax.axis_index("c")
        cbase = wid * b_per_w
        xbase = wid * c_per_w
        pltpu.sync_copy(ctcol_hbm.at[pl.ds(cbase, b_per_w)], ctcol_v)
        pltpu.sync_copy(clane_hbm.at[pl.ds(cbase, b_per_w)], clane_v)
        pltpu.sync_copy(xtcol_hbm.at[pl.ds(xbase, c_per_w)], xtcol_v)
        pltpu.sync_copy(xlane_hbm.at[pl.ds(xbase, c_per_w)], xlane_v)
        d16 = lax.iota(jnp.int32, 16)

        def extract(tile_v, acc_v, lane, g, k):
            lc16 = pl.multiple_of((lane // 16) * 16, 16)
            li = jnp.broadcast_to(lane - lc16, (16,))
            sel = d16 == k
            for h in range(DH):
                for s in range(8):
                    v = tile_v[h, s, pl.ds(lc16, 16)]
                    gv = lax.gather(
                        v, li[:, None],
                        lax.GatherDimensionNumbers(
                            offset_dims=(), collapsed_slice_dims=(0,),
                            start_index_map=(0,)),
                        (1,),
                        mode=lax.GatherScatterMode.PROMISE_IN_BOUNDS)
                    cur = acc_v[h, s, pl.ds(g * 16, 16)]
                    acc_v[h, s, pl.ds(g * 16, 16)] = jnp.where(sel, gv, cur)

        def step(g, _):
            ctcol = ctcol_v[pl.ds(g * 16, 16)]
            clane = clane_v[pl.ds(g * 16, 16)]
            xtcol = xtcol_v[pl.ds(g * 16, 16)]
            xlane = xlane_v[pl.ds(g * 16, 16)]
            for k in range(16):
                pltpu.async_copy(
                    winT_hbm.at[:, :, pl.ds(pl.multiple_of(ctcol[k], 128), 128)],
                    ctile_v, sem,
                ).wait()
                extract(ctile_v, cacc_v, clane[k], g, k)
                pltpu.async_copy(
                    woutT_hbm.at[:, :, pl.ds(pl.multiple_of(xtcol[k], 128), 128)],
                    xtile_v, sem,
                ).wait()
                extract(xtile_v, xacc_v, xlane[k], g, k)
            return 0

        lax.fori_loop(0, b_per_w // 16, step, 0)
        pltpu.sync_copy(cacc_v, outcT_hbm.at[:, :, pl.ds(cbase, b_per_w)])
        pltpu.sync_copy(xacc_v, outxT_hbm.at[:, :, pl.ds(xbase, c_per_w)])

    return gather_k


def _mm_body(cvT_ref, xvT_ref, out_ref):
    out_ref[...] = lax.dot_general(
        cvT_ref[...], xvT_ref[...],
        (((0,), (0,)), ((), ())),
        preferred_element_type=jnp.float32,
    )


def _matmul(cvT, xvT):
    BM = 512
    BN = 4096
    grid = (_B // BM, _C // BN)
    return pl.pallas_call(
        _mm_body,
        grid=grid,
        in_specs=[
            pl.BlockSpec((_DIM, BM), lambda i, j: (0, i)),
            pl.BlockSpec((_DIM, BN), lambda i, j: (0, j)),
        ],
        out_specs=pl.BlockSpec((BM, BN), lambda i, j: (i, j)),
        out_shape=jax.ShapeDtypeStruct((_B, _C), jnp.float32),
    )(cvT, xvT)


def kernel(center_words, all_context_words, W_in, W_out):
    cidx = center_words.astype(jnp.int32)
    xidx = all_context_words.astype(jnp.int32)
    ctcol = (cidx // 128) * 128
    clane = cidx % 128
    xtcol = (xidx // 128) * 128
    xlane = xidx % 128
    cvT3, xvT3 = _make_sc_gather(_VOCAB, _DIM, _B, _C)(
        W_in.T.reshape(_DIM // 8, 8, _VOCAB), ctcol, clane,
        W_out.T.reshape(_DIM // 8, 8, _VOCAB), xtcol, xlane)
    cvT = cvT3.reshape(_DIM, _B)
    xvT = xvT3.reshape(_DIM, _C)
    return _matmul(cvT, xvT)


# speedup vs baseline: 8.3410x; 2.7677x over previous
import functools

import jax
import jax.numpy as jnp
from jax import lax
from jax.experimental import pallas as pl
from jax.experimental.pallas import tpu as pltpu
from jax.experimental.pallas import tpu_sc as plsc

_VOCAB = 1000000
_DIM = 32
_B = 4096
_C = 4096


@functools.lru_cache(maxsize=None)
def _make_sc_gather(V, D, B, C):
    NC, NS = 2, 16
    NW = NC * NS
    b_per_w = B // NW
    c_per_w = C // NW
    DH = D // 8
    NB = 8
    mesh = plsc.VectorSubcoreMesh(core_axis_name="c", subcore_axis_name="s")

    @functools.partial(
        pl.kernel,
        mesh=mesh,
        out_type=[
            jax.ShapeDtypeStruct((DH, 8, B), jnp.float32),
            jax.ShapeDtypeStruct((DH, 8, C), jnp.float32),
        ],
        scratch_types=[
            pltpu.VMEM((b_per_w + 2 * NB,), jnp.int32),
            pltpu.VMEM((b_per_w + 2 * NB,), jnp.int32),
            pltpu.VMEM((c_per_w + 2 * NB,), jnp.int32),
            pltpu.VMEM((c_per_w + 2 * NB,), jnp.int32),
            pltpu.VMEM((2, NB, DH, 8, 128), jnp.float32),
            pltpu.VMEM((DH, 8, b_per_w), jnp.float32),
            pltpu.VMEM((DH, 8, c_per_w), jnp.float32),
            pltpu.SemaphoreType.DMA,
            pltpu.SemaphoreType.DMA,
        ],
    )
    def gather_k(winT_hbm, ctcol_hbm, clane_hbm, woutT_hbm, xtcol_hbm,
                 xlane_hbm, outcT_hbm, outxT_hbm,
                 ctcol_v, clane_v, xtcol_v, xlane_v,
                 slots_v, cacc_v, xacc_v, sem0, sem1):
        wid = lax.axis_index("s") * NC + lax.axis_index("c")
        cbase = wid * b_per_w
        xbase = wid * c_per_w
        pltpu.sync_copy(ctcol_hbm.at[pl.ds(cbase, b_per_w)],
                        ctcol_v.at[pl.ds(0, b_per_w)])
        pltpu.sync_copy(clane_hbm.at[pl.ds(cbase, b_per_w)],
                        clane_v.at[pl.ds(0, b_per_w)])
        pltpu.sync_copy(xtcol_hbm.at[pl.ds(xbase, c_per_w)],
                        xtcol_v.at[pl.ds(0, c_per_w)])
        pltpu.sync_copy(xlane_hbm.at[pl.ds(xbase, c_per_w)],
                        xlane_v.at[pl.ds(0, c_per_w)])
        d16 = lax.iota(jnp.int32, 16)
        sems = (sem0, sem1)

        def gather_table(tab_hbm, tcol_v, lane_v, acc_v, n_words):
            nbatch = n_words // NB

            def fire(slot, sem, off16):
                tcol = tcol_v[pl.ds(off16, 16)]
                for b in range(NB):
                    pltpu.async_copy(
                        tab_hbm.at[:, :, pl.ds(pl.multiple_of(tcol[b], 128), 128)],
                        slots_v.at[slot, b], sem,
                    )

            def drain(slot, sem):
                for b in range(NB):
                    pltpu.make_async_copy(
                        tab_hbm.at[:, :, pl.ds(0, 128)],
                        slots_v.at[slot, b], sem,
                    ).wait()

            def extract(slot, p, off16, colg16):
                lane = lane_v[pl.ds(off16, 16)]
                for h in range(DH):
                    for s in range(8):
                        cur = acc_v[h, s, pl.ds(colg16, 16)]
                        for b in range(NB):
                            lb = lane[b]
                            lc16 = pl.multiple_of((lb // 16) * 16, 16)
                            li = jnp.broadcast_to(lb - lc16, (16,))
                            v = slots_v[slot, b, h, s, pl.ds(lc16, 16)]
                            gv = lax.gather(
                                v, li[:, None],
                                lax.GatherDimensionNumbers(
                                    offset_dims=(), collapsed_slice_dims=(0,),
                                    start_index_map=(0,)),
                                (1,),
                                mode=lax.GatherScatterMode.PROMISE_IN_BOUNDS)
                            cur = jnp.where(d16 == p * NB + b, gv, cur)
                        acc_v[h, s, pl.ds(colg16, 16)] = cur

            fire(0, sems[0], 0)
            fire(1, sems[1], NB)

            def body(u, _):
                colg16 = pl.multiple_of(u * 16, 16)
                for p in range(2):
                    t = 2 * u + p
                    drain(p, sems[p])
                    extract(p, p, t * NB, colg16)

                    @pl.when(t + 2 < nbatch)
                    def _():
                        fire(p, sems[p], (t + 2) * NB)
                return 0

            lax.fori_loop(0, nbatch // 2, body, 0)

        gather_table(winT_hbm, ctcol_v, clane_v, cacc_v, b_per_w)
        gather_table(woutT_hbm, xtcol_v, xlane_v, xacc_v, c_per_w)
        pltpu.sync_copy(cacc_v, outcT_hbm.at[:, :, pl.ds(cbase, b_per_w)])
        pltpu.sync_copy(xacc_v, outxT_hbm.at[:, :, pl.ds(xbase, c_per_w)])

    return gather_k


def _mm_body(cvT_ref, xvT_ref, out_ref):
    out_ref[...] = lax.dot_general(
        cvT_ref[...], xvT_ref[...],
        (((0,), (0,)), ((), ())),
        preferred_element_type=jnp.float32,
    )


def _matmul(cvT, xvT):
    BM = 512
    BN = 4096
    grid = (_B // BM, _C // BN)
    return pl.pallas_call(
        _mm_body,
        grid=grid,
        in_specs=[
            pl.BlockSpec((_DIM, BM), lambda i, j: (0, i)),
            pl.BlockSpec((_DIM, BN), lambda i, j: (0, j)),
        ],
        out_specs=pl.BlockSpec((BM, BN), lambda i, j: (i, j)),
        out_shape=jax.ShapeDtypeStruct((_B, _C), jnp.float32),
    )(cvT, xvT)


def kernel(center_words, all_context_words, W_in, W_out):
    cidx = center_words.astype(jnp.int32)
    xidx = all_context_words.astype(jnp.int32)
    ctcol = (cidx // 128) * 128
    clane = cidx % 128
    xtcol = (xidx // 128) * 128
    xlane = xidx % 128
    cvT3, xvT3 = _make_sc_gather(_VOCAB, _DIM, _B, _C)(
        W_in.T.reshape(_DIM // 8, 8, _VOCAB), ctcol, clane,
        W_out.T.reshape(_DIM // 8, 8, _VOCAB), xtcol, xlane)
    cvT = cvT3.reshape(_DIM, _B)
    xvT = xvT3.reshape(_DIM, _C)
    return _matmul(cvT, xvT)
